# Initial kernel scaffold; baseline (speedup 1.0000x reference)
import jax
import jax.numpy as jnp
from jax.experimental import pallas as pl


def kernel(x, W_Iq, W_Ik, W_Iw, gate_bias, head_bias, variance_ema, W_q, W_k, W_v, W_gv, W_go, W_o):
    raise NotImplementedError("write your pallas kernel here")


# speedup vs baseline: 17.4399x; 17.4399x over previous
import numpy as np
import jax
import jax.numpy as jnp
from jax.experimental import pallas as pl

H = 16
IH = 4
DI = 32
HD = 64
KSEL = 512
BQ = 256

_I32_MIN = np.int32(-2**31)
_I32_MAX = np.int32(2**31 - 1)


def _rope_tables(T):
    base = 10000.0
    inv_freq = 1.0 / base ** (np.arange(0, HD, 2, dtype=np.float64) / HD)
    wavelen = 2.0 * np.pi / inv_freq
    ramp = np.clip((wavelen - 1.0) / (32.0 - 1.0), 0.0, 1.0)
    scale = 1.0 + (32.0 - 1.0) * ramp
    t = np.arange(T, dtype=np.float64)
    freqs = (t[:, None] / scale[None, :]) * inv_freq[None, :]
    emb = np.concatenate([freqs, freqs], axis=-1)
    c2 = np.cos(emb)[:, ::2].astype(np.float32)
    s2 = np.sin(emb)[:, ::2].astype(np.float32)
    return c2, s2


def _qkv_body(x_ref, wq_ref, wk_ref, wv_ref, wgv_ref, c2_ref, s2_ref,
              q_ref, k_ref, vg_ref):
    xf = x_ref[...]
    c2 = c2_ref[...]
    s2 = s2_ref[...]

    def rope(y):
        y1 = y[:, :HD // 2]
        y2 = y[:, HD // 2:]
        return jnp.concatenate([y1 * c2 - y2 * s2, y1 * s2 + y2 * c2], axis=1)

    q = jnp.dot(xf, wq_ref[0], preferred_element_type=jnp.float32)
    k = jnp.dot(xf, wk_ref[0], preferred_element_type=jnp.float32)
    v = jnp.dot(xf, wv_ref[0], preferred_element_type=jnp.float32)
    gv = jax.nn.sigmoid(jnp.dot(xf, wgv_ref[0], preferred_element_type=jnp.float32))
    q_ref[0] = rope(q)
    k_ref[0] = rope(k)
    vg_ref[0] = v * gv


def _idx_body(x_ref, wiq_ref, wik_ref, qi_ref, ki_ref):
    xf = x_ref[...]
    qi_ref[0] = jnp.dot(xf, wiq_ref[0], preferred_element_type=jnp.float32)
    ki_ref[0] = jnp.dot(xf, wik_ref[0], preferred_element_type=jnp.float32)


def _attn_body(qi_ref, ki_ref, q_ref, k_ref, vg_ref, o_ref):
    T = ki_ref.shape[1]
    it = pl.program_id(1)
    ml = jax.lax.dot_general(qi_ref[0], ki_ref[0], (((1,), (1,)), ((), ())),
                             preferred_element_type=jnp.float32)
    u = jax.lax.bitcast_convert_type(ml, jnp.int32)
    key = jnp.where(u < 0, u ^ jnp.int32(0x7FFFFFFF), u)
    rows = it * BQ + jax.lax.broadcasted_iota(jnp.int32, (BQ, T), 0)
    cols = jax.lax.broadcasted_iota(jnp.int32, (BQ, T), 1)
    causal = cols <= rows
    keym = jnp.where(causal, key, _I32_MIN)

    def body(_, lohi):
        lo, hi = lohi
        mid = (lo >> 1) + (hi >> 1) + ((lo & 1) | (hi & 1))
        cnt = jnp.sum((keym >= mid).astype(jnp.int32), axis=1, keepdims=True)
        p = cnt >= KSEL
        return jnp.where(p, mid, lo), jnp.where(p, hi, mid - 1)

    lo0 = jnp.full((BQ, 1), _I32_MIN, jnp.int32)
    hi0 = jnp.full((BQ, 1), _I32_MAX, jnp.int32)
    lo, _ = jax.lax.fori_loop(0, 32, body, (lo0, hi0))
    sel = causal & (keym >= lo)

    outs = []
    for j in range(H // IH):
        logit = jax.lax.dot_general(q_ref[j], k_ref[j], (((1,), (1,)), ((), ())),
                                    preferred_element_type=jnp.float32)
        logit = jnp.where(sel, logit * jnp.float32(0.125), jnp.float32(-1e30))
        m = jnp.max(logit, axis=1, keepdims=True)
        e = jnp.exp(logit - m)
        a = e / jnp.sum(e, axis=1, keepdims=True)
        outs.append(jnp.dot(a, vg_ref[j], preferred_element_type=jnp.float32))
    o_ref[...] = jnp.concatenate(outs, axis=1)


def _out_body(x_ref, ao_ref, wgo_ref, wo_ref, o_ref):
    go = jax.nn.sigmoid(jnp.dot(x_ref[...], wgo_ref[...],
                                preferred_element_type=jnp.float32))
    o_ref[...] = jnp.dot(ao_ref[...] * go, wo_ref[...],
                         preferred_element_type=jnp.float32)


def kernel(x, W_Iq, W_Ik, W_Iw, gate_bias, head_bias, variance_ema,
           W_q, W_k, W_v, W_gv, W_go, W_o):
    B, T, C = x.shape
    NT = T // BQ
    xf = x[0]
    f32 = jnp.float32

    perm = np.concatenate([np.arange(0, HD, 2), np.arange(1, HD, 2)])
    wq = W_q.reshape(C, H, HD)[:, :, perm].transpose(1, 0, 2)
    wk = W_k.reshape(C, H, HD)[:, :, perm].transpose(1, 0, 2)
    wv = W_v.reshape(C, H, HD).transpose(1, 0, 2)
    wgv = W_gv.reshape(C, H, HD).transpose(1, 0, 2)
    wiq = W_Iq.reshape(C, IH, DI).transpose(1, 0, 2)
    wik = W_Ik.reshape(C, IH, DI).transpose(1, 0, 2)
    c2np, s2np = _rope_tables(T)
    c2 = jnp.asarray(c2np)
    s2 = jnp.asarray(s2np)

    full2 = lambda a, b: pl.BlockSpec((a, b), lambda *_: (0, 0))
    q, k, vg = pl.pallas_call(
        _qkv_body,
        grid=(H,),
        in_specs=[
            full2(T, C),
            pl.BlockSpec((1, C, HD), lambda h: (h, 0, 0)),
            pl.BlockSpec((1, C, HD), lambda h: (h, 0, 0)),
            pl.BlockSpec((1, C, HD), lambda h: (h, 0, 0)),
            pl.BlockSpec((1, C, HD), lambda h: (h, 0, 0)),
            full2(T, HD // 2),
            full2(T, HD // 2),
        ],
        out_specs=[pl.BlockSpec((1, T, HD), lambda h: (h, 0, 0))] * 3,
        out_shape=[jax.ShapeDtypeStruct((H, T, HD), f32)] * 3,
    )(xf, wq, wk, wv, wgv, c2, s2)

    qi, ki = pl.pallas_call(
        _idx_body,
        grid=(IH,),
        in_specs=[
            full2(T, C),
            pl.BlockSpec((1, C, DI), lambda h: (h, 0, 0)),
            pl.BlockSpec((1, C, DI), lambda h: (h, 0, 0)),
        ],
        out_specs=[pl.BlockSpec((1, T, DI), lambda h: (h, 0, 0))] * 2,
        out_shape=[jax.ShapeDtypeStruct((IH, T, DI), f32)] * 2,
    )(xf, wiq, wik)

    ao = pl.pallas_call(
        _attn_body,
        grid=(IH, NT),
        in_specs=[
            pl.BlockSpec((1, BQ, DI), lambda ih, it: (ih, it, 0)),
            pl.BlockSpec((1, T, DI), lambda ih, it: (ih, 0, 0)),
            pl.BlockSpec((H // IH, BQ, HD), lambda ih, it: (ih, it, 0)),
            pl.BlockSpec((H // IH, T, HD), lambda ih, it: (ih, 0, 0)),
            pl.BlockSpec((H // IH, T, HD), lambda ih, it: (ih, 0, 0)),
        ],
        out_specs=pl.BlockSpec((BQ, (H // IH) * HD), lambda ih, it: (it, ih)),
        out_shape=jax.ShapeDtypeStruct((T, C), f32),
    )(qi, ki, q, k, vg)

    out = pl.pallas_call(
        _out_body,
        grid=(NT,),
        in_specs=[
            pl.BlockSpec((BQ, C), lambda it: (it, 0)),
            pl.BlockSpec((BQ, C), lambda it: (it, 0)),
            full2(C, C),
            full2(C, C),
        ],
        out_specs=pl.BlockSpec((BQ, C), lambda it: (it, 0)),
        out_shape=jax.ShapeDtypeStruct((T, C), f32),
    )(xf, ao, W_go, W_o)

    return out[None]
